# baseline (speedup 1.0000x reference)
import functools

import jax
import jax.numpy as jnp
from jax import lax
from jax.experimental import pallas as pl
from jax.experimental.pallas import tpu as pltpu
from jax.experimental.pallas import tpu_sc as plsc

_S = 30.0
_B = 16384
_C = 100
_CP = 112
_NC, _NS, _L = 2, 16, 16
_NW = _NC * _NS
_PER_W = _B // _NW
_VPER = _PER_W // _L
_RB = 8192


def _sc_margin_body(tgt_hbm, mlist_hbm, mb_hbm, tgt_v, mlist_v, mb_v):
    wid = lax.axis_index("s") * _NC + lax.axis_index("c")
    base = wid * _PER_W
    pltpu.sync_copy(tgt_hbm.at[pl.ds(base, _PER_W)], tgt_v)
    pltpu.sync_copy(mlist_hbm, mlist_v)
    groups = [mlist_v[pl.ds(g * _L, _L)] for g in range(_CP // _L)]
    for j in range(_VPER):
        t16 = tgt_v[pl.ds(j * _L, _L)]
        lane = lax.bitwise_and(t16, 15)
        grp = lax.shift_right_logical(t16, 4)
        res = jnp.zeros((_L,), jnp.float32)
        for g in range(_CP // _L):
            gv = groups[g].at[lane].get(mode="promise_in_bounds")
            res = jnp.where(grp == g, gv, res)
        mb_v[pl.ds(j * _L, _L)] = res
    pltpu.sync_copy(mb_v, mb_hbm.at[pl.ds(base, _PER_W)])


@functools.cache
def _sc_margin_kernel():
    return pl.kernel(
        _sc_margin_body,
        out_type=jax.ShapeDtypeStruct((_B,), jnp.float32),
        mesh=plsc.VectorSubcoreMesh(core_axis_name="c", subcore_axis_name="s",
                                    num_cores=_NC, num_subcores=_NS),
        scratch_types=[
            pltpu.VMEM((_PER_W,), jnp.int32),
            pltpu.VMEM((_CP,), jnp.float32),
            pltpu.VMEM((_PER_W,), jnp.float32),
        ],
    )


def _tc_loss_body(x_ref, tgt_ref, mb_ref, o_ref):
    y = x_ref[...] * _S
    col = lax.broadcasted_iota(jnp.int32, (_RB, _C), 1)
    mask = col == tgt_ref[...]
    yadj = jnp.where(mask, y - mb_ref[...] * _S, y)
    m = jnp.max(yadj, axis=1, keepdims=True)
    s1 = jnp.sum(jnp.exp(yadj - m), axis=1, keepdims=True)
    tl = jnp.sum(jnp.where(mask, yadj, 0.0), axis=1, keepdims=True)
    part = jnp.sum(m + jnp.log(s1) - tl, keepdims=True).reshape(1, 1)

    @pl.when(pl.program_id(0) == 0)
    def _():
        o_ref[...] = jnp.zeros((1, 1), jnp.float32)

    o_ref[...] += part * (1.0 / _B)


_tc_loss = pl.pallas_call(
    _tc_loss_body,
    grid=(_B // _RB,),
    in_specs=[pl.BlockSpec((_RB, _C), lambda i: (i, 0)),
              pl.BlockSpec((_RB, 1), lambda i: (i, 0)),
              pl.BlockSpec((_RB, 1), lambda i: (i, 0))],
    out_specs=pl.BlockSpec((1, 1), lambda i: (0, 0)),
    out_shape=jax.ShapeDtypeStruct((1, 1), jnp.float32),
)


def kernel(x, target, m_list):
    tgt = target.astype(jnp.int32)
    mlist_pad = jnp.pad(m_list, (0, _CP - _C))
    mb = _sc_margin_kernel()(tgt, mlist_pad)
    loss = _tc_loss(x, tgt.reshape(_B, 1), mb.reshape(_B, 1))
    return loss[0, 0]


# speedup vs baseline: 1.0039x; 1.0039x over previous
import functools

import jax
import jax.numpy as jnp
from jax import lax
from jax.experimental import pallas as pl
from jax.experimental.pallas import tpu as pltpu
from jax.experimental.pallas import tpu_sc as plsc

_S = 30.0
_B = 16384
_C = 100
_CP = 112
_NC, _NS, _L = 2, 16, 16
_NW = _NC * _NS
_PER_W = _B // _NW
_VPER = _PER_W // _L
_RB = 2048


def _sc_margin_body(tgt_hbm, mlist_hbm, mb_hbm, tgt_v, mlist_v, mb_v):
    wid = lax.axis_index("s") * _NC + lax.axis_index("c")
    base = wid * _PER_W
    pltpu.sync_copy(tgt_hbm.at[pl.ds(base, _PER_W)], tgt_v)
    pltpu.sync_copy(mlist_hbm, mlist_v)
    groups = [mlist_v[pl.ds(g * _L, _L)] for g in range(_CP // _L)]
    for j in range(_VPER):
        t16 = tgt_v[pl.ds(j * _L, _L)]
        lane = lax.bitwise_and(t16, 15)
        grp = lax.shift_right_logical(t16, 4)
        res = jnp.zeros((_L,), jnp.float32)
        for g in range(_CP // _L):
            gv = groups[g].at[lane].get(mode="promise_in_bounds")
            res = jnp.where(grp == g, gv, res)
        mb_v[pl.ds(j * _L, _L)] = res
    pltpu.sync_copy(mb_v, mb_hbm.at[pl.ds(base, _PER_W)])


@functools.cache
def _sc_margin_kernel():
    return pl.kernel(
        _sc_margin_body,
        out_type=jax.ShapeDtypeStruct((_B,), jnp.float32),
        mesh=plsc.VectorSubcoreMesh(core_axis_name="c", subcore_axis_name="s",
                                    num_cores=_NC, num_subcores=_NS),
        scratch_types=[
            pltpu.VMEM((_PER_W,), jnp.int32),
            pltpu.VMEM((_CP,), jnp.float32),
            pltpu.VMEM((_PER_W,), jnp.float32),
        ],
    )


def _tc_loss_body(x_ref, tgt_ref, mb_ref, o_ref):
    y = x_ref[...] * _S
    col = lax.broadcasted_iota(jnp.int32, (_RB, _C), 1)
    mask = col == tgt_ref[...]
    yadj = jnp.where(mask, y - mb_ref[...] * _S, y)
    m = jnp.max(yadj, axis=1, keepdims=True)
    s1 = jnp.sum(jnp.exp(yadj - m), axis=1, keepdims=True)
    tl = jnp.sum(jnp.where(mask, yadj, 0.0), axis=1, keepdims=True)
    part = jnp.sum(m + jnp.log(s1) - tl, keepdims=True).reshape(1, 1)

    @pl.when(pl.program_id(0) == 0)
    def _():
        o_ref[...] = jnp.zeros((1, 1), jnp.float32)

    o_ref[...] += part * (1.0 / _B)


_tc_loss = pl.pallas_call(
    _tc_loss_body,
    grid=(_B // _RB,),
    in_specs=[pl.BlockSpec((_RB, _C), lambda i: (i, 0)),
              pl.BlockSpec((_RB, 1), lambda i: (i, 0)),
              pl.BlockSpec((_RB, 1), lambda i: (i, 0))],
    out_specs=pl.BlockSpec((1, 1), lambda i: (0, 0)),
    out_shape=jax.ShapeDtypeStruct((1, 1), jnp.float32),
)


def kernel(x, target, m_list):
    tgt = target.astype(jnp.int32)
    mlist_pad = jnp.pad(m_list, (0, _CP - _C))
    mb = _sc_margin_kernel()(tgt, mlist_pad)
    loss = _tc_loss(x, tgt.reshape(_B, 1), mb.reshape(_B, 1))
    return loss[0, 0]


# speedup vs baseline: 1.0639x; 1.0598x over previous
import functools

import jax
import jax.numpy as jnp
from jax import lax
from jax.experimental import pallas as pl
from jax.experimental.pallas import tpu as pltpu
from jax.experimental.pallas import tpu_sc as plsc

_S = 30.0
_B = 16384
_C = 100
_CP = 112
_NC, _NS, _L = 2, 16, 16
_NW = _NC * _NS
_PER_W = _B // _NW
_VPER = _PER_W // _L
_RB = 4096


def _sc_margin_body(tgt_hbm, mlist_hbm, mb_hbm, tgt_v, mlist_v, mb_v):
    wid = lax.axis_index("s") * _NC + lax.axis_index("c")
    base = wid * _PER_W
    pltpu.sync_copy(tgt_hbm.at[pl.ds(base, _PER_W)], tgt_v)
    pltpu.sync_copy(mlist_hbm, mlist_v)
    groups = [mlist_v[pl.ds(g * _L, _L)] for g in range(_CP // _L)]
    for j in range(_VPER):
        t16 = tgt_v[pl.ds(j * _L, _L)]
        lane = lax.bitwise_and(t16, 15)
        grp = lax.shift_right_logical(t16, 4)
        res = jnp.zeros((_L,), jnp.float32)
        for g in range(_CP // _L):
            gv = groups[g].at[lane].get(mode="promise_in_bounds")
            res = jnp.where(grp == g, gv, res)
        mb_v[pl.ds(j * _L, _L)] = res
    pltpu.sync_copy(mb_v, mb_hbm.at[pl.ds(base, _PER_W)])


@functools.cache
def _sc_margin_kernel():
    return pl.kernel(
        _sc_margin_body,
        out_type=jax.ShapeDtypeStruct((_B,), jnp.float32),
        mesh=plsc.VectorSubcoreMesh(core_axis_name="c", subcore_axis_name="s",
                                    num_cores=_NC, num_subcores=_NS),
        scratch_types=[
            pltpu.VMEM((_PER_W,), jnp.int32),
            pltpu.VMEM((_CP,), jnp.float32),
            pltpu.VMEM((_PER_W,), jnp.float32),
        ],
    )


def _tc_loss_body(x_ref, tgt_ref, mb_ref, o_ref):
    y = x_ref[...] * _S
    col = lax.broadcasted_iota(jnp.int32, (_RB, _C), 1)
    mask = col == tgt_ref[...]
    yadj = jnp.where(mask, y - mb_ref[...] * _S, y)
    m = jnp.max(yadj, axis=1, keepdims=True)
    e = jnp.exp(yadj - m)
    masked = jnp.where(mask, yadj, 0.0)
    ones = jnp.ones((_C, 1), jnp.float32)
    s1 = jnp.dot(e, ones, preferred_element_type=jnp.float32)
    tl = jnp.dot(masked, ones, preferred_element_type=jnp.float32)
    part = jnp.sum(m + jnp.log(s1) - tl, keepdims=True).reshape(1, 1)

    @pl.when(pl.program_id(0) == 0)
    def _():
        o_ref[...] = jnp.zeros((1, 1), jnp.float32)

    o_ref[...] += part * (1.0 / _B)


_tc_loss = pl.pallas_call(
    _tc_loss_body,
    grid=(_B // _RB,),
    in_specs=[pl.BlockSpec((_RB, _C), lambda i: (i, 0)),
              pl.BlockSpec((_RB, 1), lambda i: (i, 0)),
              pl.BlockSpec((_RB, 1), lambda i: (i, 0))],
    out_specs=pl.BlockSpec((1, 1), lambda i: (0, 0)),
    out_shape=jax.ShapeDtypeStruct((1, 1), jnp.float32),
)


def kernel(x, target, m_list):
    tgt = target.astype(jnp.int32)
    mlist_pad = jnp.pad(m_list, (0, _CP - _C))
    mb = _sc_margin_kernel()(tgt, mlist_pad)
    loss = _tc_loss(x, tgt.reshape(_B, 1), mb.reshape(_B, 1))
    return loss[0, 0]


# speedup vs baseline: 1.0680x; 1.0039x over previous
import functools

import jax
import jax.numpy as jnp
from jax import lax
from jax.experimental import pallas as pl
from jax.experimental.pallas import tpu as pltpu
from jax.experimental.pallas import tpu_sc as plsc

_S = 30.0
_B = 16384
_C = 100
_CP = 112
_NC, _NS, _L = 2, 16, 16
_NW = _NC * _NS
_PER_W = _B // _NW
_VPER = _PER_W // _L
_RB = 4096


def _sc_margin_body(tgt_hbm, mlist_hbm, mb_hbm, tgt_v, mlist_v, mb_v):
    wid = lax.axis_index("s") * _NC + lax.axis_index("c")
    base = wid * _PER_W
    pltpu.sync_copy(tgt_hbm.at[pl.ds(base, _PER_W)], tgt_v)
    pltpu.sync_copy(mlist_hbm, mlist_v)
    groups = [mlist_v[pl.ds(g * _L, _L)] for g in range(_CP // _L)]
    for j in range(_VPER):
        t16 = tgt_v[pl.ds(j * _L, _L)]
        lane = lax.bitwise_and(t16, 15)
        grp = lax.shift_right_logical(t16, 4)
        res = jnp.zeros((_L,), jnp.float32)
        for g in range(_CP // _L):
            gv = groups[g].at[lane].get(mode="promise_in_bounds")
            res = jnp.where(grp == g, gv, res)
        mb_v[pl.ds(j * _L, _L)] = res
    pltpu.sync_copy(mb_v, mb_hbm.at[pl.ds(base, _PER_W)])


@functools.cache
def _sc_margin_kernel():
    return pl.kernel(
        _sc_margin_body,
        out_type=jax.ShapeDtypeStruct((_B,), jnp.float32),
        mesh=plsc.VectorSubcoreMesh(core_axis_name="c", subcore_axis_name="s",
                                    num_cores=_NC, num_subcores=_NS),
        scratch_types=[
            pltpu.VMEM((_PER_W,), jnp.int32),
            pltpu.VMEM((_CP,), jnp.float32),
            pltpu.VMEM((_PER_W,), jnp.float32),
        ],
    )


def _tc_loss_body(x_ref, tgt_ref, mb_ref, o_ref):
    y = x_ref[...] * _S
    col = lax.broadcasted_iota(jnp.int32, (_RB, _C), 1)
    mask = col == tgt_ref[...]
    yadj = jnp.where(mask, y - mb_ref[...] * _S, y)
    m = jnp.max(yadj, axis=1, keepdims=True)
    e = jnp.exp(yadj - m)
    masked = jnp.where(mask, yadj, 0.0)
    ones = jnp.ones((_C, 1), jnp.float32)
    s1 = jnp.dot(e, ones, preferred_element_type=jnp.float32)
    tl = jnp.dot(masked, ones, preferred_element_type=jnp.float32)
    ones_r = jnp.ones((1, _RB), jnp.float32)
    part = jnp.dot(ones_r, m + jnp.log(s1) - tl,
                   preferred_element_type=jnp.float32)

    @pl.when(pl.program_id(0) == 0)
    def _():
        o_ref[...] = jnp.zeros((1, 1), jnp.float32)

    o_ref[...] += part * (1.0 / _B)


_tc_loss = pl.pallas_call(
    _tc_loss_body,
    grid=(_B // _RB,),
    in_specs=[pl.BlockSpec((_RB, _C), lambda i: (i, 0)),
              pl.BlockSpec((_RB, 1), lambda i: (i, 0)),
              pl.BlockSpec((_RB, 1), lambda i: (i, 0))],
    out_specs=pl.BlockSpec((1, 1), lambda i: (0, 0)),
    out_shape=jax.ShapeDtypeStruct((1, 1), jnp.float32),
)


def kernel(x, target, m_list):
    tgt = target.astype(jnp.int32)
    mlist_pad = jnp.pad(m_list, (0, _CP - _C))
    mb = _sc_margin_kernel()(tgt, mlist_pad)
    loss = _tc_loss(x, tgt.reshape(_B, 1), mb.reshape(_B, 1))
    return loss[0, 0]
